# Initial kernel scaffold; baseline (speedup 1.0000x reference)
import jax
import jax.numpy as jnp
from jax.experimental import pallas as pl


def kernel(user_ids, item_ids, ue_gmf, ie_gmf, ue_mlp, ie_mlp, W1, b1, W2, b2, W3, b3, Wo, bo):
    raise NotImplementedError("write your pallas kernel here")


# speedup vs baseline: 2.1122x; 2.1122x over previous
import functools

import jax
import jax.numpy as jnp
from jax import lax
from jax.experimental import pallas as pl
from jax.experimental.pallas import tpu as pltpu
from jax.experimental.pallas import tpu_sc as plsc

B = 16384
D = 128
NC = 2
NS = 16
NW = NC * NS
BPW = B // NW
CH = 128
NCH = BPW // CH


def _gather_body(uid_ref, iid_ref, ug_t, ig_t, um_t, im_t,
                 ug_o, ig_o, um_o, im_o,
                 uidx_v, iidx_v, rows_v, sem):
    wid = lax.axis_index("s") * NC + lax.axis_index("c")
    base = wid * BPW
    pltpu.sync_copy(uid_ref.at[wid], uidx_v)
    pltpu.sync_copy(iid_ref.at[wid], iidx_v)
    for table, idx_v, out in ((ug_t, uidx_v, ug_o), (ig_t, iidx_v, ig_o),
                              (um_t, uidx_v, um_o), (im_t, iidx_v, im_o)):
        copies = [
            pltpu.async_copy(table.at[idx_v.at[j]],
                             rows_v.at[pl.ds(j * CH, CH)], sem)
            for j in range(NCH)
        ]
        for c in copies:
            c.wait()
        pltpu.sync_copy(rows_v, out.at[pl.ds(base, BPW)])


def _sc_gather(user_ids, item_ids, ue_gmf, ie_gmf, ue_mlp, ie_mlp):
    mesh = plsc.VectorSubcoreMesh(core_axis_name="c", subcore_axis_name="s",
                                  num_cores=NC, num_subcores=NS)
    f = pl.kernel(
        _gather_body,
        out_type=[jax.ShapeDtypeStruct((B, D), jnp.float32)] * 4,
        mesh=mesh,
        scratch_types=[
            pltpu.VMEM((NCH, CH), jnp.int32),
            pltpu.VMEM((NCH, CH), jnp.int32),
            pltpu.VMEM((BPW, D), jnp.float32),
            pltpu.SemaphoreType.DMA,
        ],
    )
    uid = user_ids.astype(jnp.int32).reshape(NW, NCH, CH)
    iid = item_ids.astype(jnp.int32).reshape(NW, NCH, CH)
    return f(uid, iid, ue_gmf, ie_gmf, ue_mlp, ie_mlp)


BB = 512


def _mlp_body(ug, ig, um, im, w1u, w1i, b1, w2, b2, w3, b3, wog, woh, bo,
              out):
    dot = functools.partial(jnp.dot, preferred_element_type=jnp.float32)
    h = dot(um[...], w1u[...]) + dot(im[...], w1i[...]) + b1[...]
    h = jnp.maximum(h, 0.0)
    h = jnp.maximum(dot(h, w2[...]) + b2[...], 0.0)
    h = jnp.maximum(dot(h, w3[...]) + b3[...], 0.0)
    pred = dot(ug[...] * ig[...], wog[...]) + dot(h, woh[...]) + bo[0, 0]
    out[...] = pred


def _tc_mlp(ug, ig, um, im, W1, b1, W2, b2, W3, b3, Wo, bo):
    row = lambda i: (i, 0)
    zero = lambda i: (0, 0)
    rows_spec = pl.BlockSpec((BB, D), row)
    out = pl.pallas_call(
        _mlp_body,
        grid=(B // BB,),
        in_specs=[
            rows_spec, rows_spec, rows_spec, rows_spec,
            pl.BlockSpec((D, 256), zero),
            pl.BlockSpec((D, 256), zero),
            pl.BlockSpec((1, 256), zero),
            pl.BlockSpec((256, 128), zero),
            pl.BlockSpec((1, 128), zero),
            pl.BlockSpec((128, 64), zero),
            pl.BlockSpec((1, 64), zero),
            pl.BlockSpec((D, 1), zero),
            pl.BlockSpec((64, 1), zero),
            pl.BlockSpec((1, 1), zero),
        ],
        out_specs=pl.BlockSpec((BB, 1), row),
        out_shape=jax.ShapeDtypeStruct((B, 1), jnp.float32),
        compiler_params=pltpu.CompilerParams(
            dimension_semantics=("arbitrary",)),
    )(ug, ig, um, im, W1[:D], W1[D:], b1.reshape(1, 256), W2,
      b2.reshape(1, 128), W3, b3.reshape(1, 64), Wo[:D], Wo[D:],
      bo.reshape(1, 1))
    return out[:, 0]


def kernel(user_ids, item_ids, ue_gmf, ie_gmf, ue_mlp, ie_mlp,
           W1, b1, W2, b2, W3, b3, Wo, bo):
    ug, ig, um, im = _sc_gather(user_ids, item_ids, ue_gmf, ie_gmf,
                                ue_mlp, ie_mlp)
    return _tc_mlp(ug, ig, um, im, W1, b1, W2, b2, W3, b3, Wo, bo)
